# baseline (speedup 1.0000x reference)
import functools

import jax
import jax.numpy as jnp
from jax import lax
from jax.experimental import pallas as pl
from jax.experimental.pallas import tpu as pltpu
from jax.experimental.pallas import tpu_sc as plsc

N = 50000
E = 800000

_INTERPRET = False

TE = 1280
NC, NS = 2, 16
NW = NC * NS
PER_W = E // NW
GC = 1000


def _silu(x):
    return x * jax.nn.sigmoid(x)


def _gather_add(tab_r, tab_c, row, col):
    P = tab_r.shape[1]
    mesh = plsc.VectorSubcoreMesh(core_axis_name="c", subcore_axis_name="s")

    @functools.partial(
        pl.kernel,
        out_type=jax.ShapeDtypeStruct((E, P), jnp.float32),
        mesh=mesh,
        scratch_types=[
            pltpu.VMEM((GC,), jnp.int32),
            pltpu.VMEM((GC,), jnp.int32),
            pltpu.VMEM((GC, P), jnp.float32),
            pltpu.SemaphoreType.DMA,
        ],
    )
    def k(tab_r_hbm, tab_c_hbm, row_hbm, col_hbm, out_hbm, ridx, cidx, buf, sem):
        wid = lax.axis_index("s") * NC + lax.axis_index("c")
        base = wid * PER_W

        def body(i, carry):
            off = base + i * GC
            pltpu.sync_copy(row_hbm.at[pl.ds(off, GC)], ridx)
            pltpu.sync_copy(col_hbm.at[pl.ds(off, GC)], cidx)
            pltpu.async_copy(tab_r_hbm.at[ridx], buf, sem).wait()
            pltpu.async_copy(tab_c_hbm.at[cidx], buf, sem, add=True).wait()
            pltpu.sync_copy(buf, out_hbm.at[pl.ds(off, GC)])
            return carry

        lax.fori_loop(0, PER_W // GC, body, 0)

    return k(tab_r, tab_c, row, col)


CSZ = 5000
N32 = 51200
ACCW = N32 * 32
SSTRIPE = ACCW // NS


def _sc_scatter(mtflat, eflat, jobs0, jobs1, nslots):
    mesh = plsc.VectorSubcoreMesh(core_axis_name="c", subcore_axis_name="s")

    @functools.partial(
        pl.kernel,
        out_type=jax.ShapeDtypeStruct((nslots, NS, SSTRIPE), jnp.float32),
        mesh=mesh,
        scratch_types=[
            pltpu.VMEM((CSZ,), jnp.int32),
            pltpu.VMEM((CSZ,), jnp.float32),
            pltpu.VMEM_SHARED((ACCW,), jnp.float32),
            pltpu.SemaphoreType.DMA,
        ],
    )
    def k(mt_hbm, ef_hbm, z_hbm, out_hbm, ibuf, dbuf, acc, sem):
        cc = lax.axis_index("c")
        s = lax.axis_index("s")

        def run_jobs(jobs):
            for (q, lo, hi, slot) in jobs:
                ept = (hi - lo) // NS
                pltpu.sync_copy(z_hbm.at[s], acc.at[pl.ds(s * SSTRIPE, SSTRIPE)])
                plsc.subcore_barrier()

                def jbody(j, carry):
                    def body(i, carry2):
                        eo = lo + s * ept + i * CSZ
                        pltpu.sync_copy(ef_hbm.at[pl.ds(j * E + eo, CSZ)], ibuf)
                        pltpu.sync_copy(
                            mt_hbm.at[pl.ds((q * 32 + j) * E + eo, CSZ)], dbuf)
                        pltpu.async_copy(dbuf, acc.at[ibuf], sem, add=True).wait()
                        return carry2

                    lax.fori_loop(0, ept // CSZ, body, 0)
                    return carry

                lax.fori_loop(0, 32, jbody, 0)
                plsc.subcore_barrier()
                pltpu.sync_copy(acc.at[pl.ds(s * SSTRIPE, SSTRIPE)],
                                out_hbm.at[slot, s])
                plsc.subcore_barrier()

        @pl.when(cc == 0)
        def _():
            run_jobs(jobs0)

        @pl.when(cc == 1)
        def _():
            run_jobs(jobs1)

    return k(mtflat, eflat, jnp.zeros((NS, SSTRIPE), jnp.float32))


def _make_eflat(row):
    ef = row[None, :] * 32 + jnp.arange(32, dtype=jnp.int32)[:, None]
    return ef.reshape(-1)


def _combine_slots(out, hid):
    a = out.reshape(out.shape[0], N32, 32)[:, :N, :]
    if hid == 16:
        agg = a[0] + a[1]
    elif hid == 32:
        agg = jnp.concatenate([a[0], a[1]], axis=1)
    else:
        agg = jnp.concatenate([a[0], a[1], a[2] + a[3]], axis=1)
    magg = agg[:, :hid]
    trans = agg[:, hid:hid + 3]
    cnt = agg[:, hid + 3]
    return magg, trans, cnt


def _edge_kernel(hid, NB, g_ref, ea_ref, wre_ref,
                 We2_ref, be2_ref, Wc1_ref, bc1_ref, Wc2_ref,
                 mt_ref):
    g = g_ref[...]
    pre = g[:, :hid]
    cd = g[:, hid:hid + 3]
    radial = jnp.sum(cd * cd, axis=1, keepdims=True)
    ea = ea_ref[...]
    rad_ea = jnp.concatenate([radial, ea], axis=1)
    pre = pre + jnp.dot(rad_ea, wre_ref[...], preferred_element_type=jnp.float32)
    m = _silu(pre)
    m = _silu(jnp.dot(m, We2_ref[...], preferred_element_type=jnp.float32)
              + be2_ref[...])
    tt = _silu(jnp.dot(m, Wc1_ref[...], preferred_element_type=jnp.float32)
               + bc1_ref[...])
    t = jnp.dot(tt, Wc2_ref[...], preferred_element_type=jnp.float32)
    ones = jnp.ones_like(t)
    pad = jnp.zeros((m.shape[0], NB * 32 - hid - 4), jnp.float32)
    mt = jnp.concatenate([m, cd * t, ones, pad], axis=1)
    for q in range(NB):
        mt_ref[q] = jnp.transpose(mt[:, 32 * q:32 * (q + 1)])


def _edge_mlp(g, edge_attr, p, NB):
    hid = p['We2'].shape[0]
    inf = (p['We1'].shape[0] - 2) // 2
    wre = p['We1'][2 * inf:]
    P = g.shape[1]
    grid = (E // TE,)
    erow = lambda i: (i, 0)
    wfull = lambda i: (0, 0)
    out = pl.pallas_call(
        functools.partial(_edge_kernel, hid, NB),
        grid=grid,
        in_specs=[
            pl.BlockSpec((TE, P), erow),
            pl.BlockSpec((TE, 1), erow),
            pl.BlockSpec(wre.shape, wfull),
            pl.BlockSpec(p['We2'].shape, wfull),
            pl.BlockSpec((1, hid), wfull),
            pl.BlockSpec(p['Wc1'].shape, wfull),
            pl.BlockSpec((1, hid), wfull),
            pl.BlockSpec(p['Wc2'].shape, wfull),
        ],
        out_specs=[
            pl.BlockSpec((NB, 32, TE), lambda i: (0, 0, i)),
        ],
        out_shape=[
            jax.ShapeDtypeStruct((NB, 32, E), jnp.float32),
        ],
        interpret=_INTERPRET,
    )(g, edge_attr,
      wre, p['We2'], p['be2'][None, :], p['Wc1'], p['bc1'][None, :], p['Wc2'])
    return out[0].reshape(-1)


def _segment_sum(data, seg, num):
    return jax.ops.segment_sum(data, seg, num_segments=num)


def kernel(pos, edge_attr, params, edge_index, face, vertex2face, batch, ptr,
           face_len, vertex2face_len):
    row, col = edge_index[0], edge_index[1]

    centroid = jnp.mean(pos, axis=0, keepdims=True)
    p = pos - centroid
    mx = jnp.max(jnp.sqrt(jnp.sum(p ** 2, axis=1)))
    p = p / mx

    v0 = p[face[0]]
    v1 = p[face[1]]
    v2 = p[face[2]]
    fn = jnp.cross(v1 - v0, v2 - v0)
    face_area = jnp.sqrt(jnp.sum(fn ** 2, axis=1)) / 2.0
    vtx = vertex2face[:, 0]
    fidx = vertex2face[:, 1]
    asum = _segment_sum(face_area[fidx], vtx, N)
    acnt = jnp.maximum(_segment_sum(jnp.ones((vtx.shape[0],), jnp.float32), vtx, N), 1.0)
    area = asum / acnt
    x = area[:, None] * params['feat_W'][0][None, :] + params['feat_b'][None, :]

    coord = p
    eflat = _make_eflat(row)
    for lp in (params['c1'], params['c2'], params['c3']):
        hid = lp['We2'].shape[0]
        inf = (lp['We1'].shape[0] - 2) // 2
        A = lp['We1'][:inf]
        B = lp['We1'][inf:2 * inf]
        P = 128
        pad = jnp.zeros((N, P - hid - 3), jnp.float32)
        tab_r = jnp.concatenate([x @ A + lp['be1'][None, :], coord, pad], axis=1)
        tab_c = jnp.concatenate([x @ B, -coord, pad], axis=1)
        g = _gather_add(tab_r, tab_c, row, col)
        NB = (hid + 4 + 31) // 32
        mtflat = _edge_mlp(g, edge_attr, lp, NB)
        if NB == 1:
            jobs0 = ((0, 0, E // 2, 0),)
            jobs1 = ((0, E // 2, E, 1),)
            nslots = 2
        elif NB == 2:
            jobs0 = ((0, 0, E, 0),)
            jobs1 = ((1, 0, E, 1),)
            nslots = 2
        else:
            jobs0 = ((0, 0, E, 0), (2, 0, E // 2, 2))
            jobs1 = ((1, 0, E, 1), (2, E // 2, E, 3))
            nslots = 4
        out = _sc_scatter(mtflat, eflat, jobs0, jobs1, nslots)
        magg, trans, cnt = _combine_slots(out, hid)
        cnt = jnp.maximum(cnt, 1.0)
        coord = coord + trans / cnt[:, None]
        h = jnp.concatenate([x, magg], axis=1)
        h = _silu(h @ lp['Wn1'] + lp['bn1'])
        x = h @ lp['Wn2'] + lp['bn2']

    x = jax.nn.relu(x @ params['lin1_W'] + params['lin1_b'])
    x = jnp.mean(x, axis=0, keepdims=True)
    x = x @ params['lin2_W'] + params['lin2_b']
    return jax.nn.log_softmax(x, axis=1)


# speedup vs baseline: 1.7719x; 1.7719x over previous
import functools

import jax
import jax.numpy as jnp
from jax import lax
from jax.experimental import pallas as pl
from jax.experimental.pallas import tpu as pltpu
from jax.experimental.pallas import tpu_sc as plsc

N = 50000
E = 800000

_INTERPRET = False

TE = 1280
NC, NS = 2, 16
NW = NC * NS
PER_W = E // NW
GC = 1000


def _silu(x):
    return x * jax.nn.sigmoid(x)


def _gather_add(tab_r, tab_c, row, col):
    P = tab_r.shape[1]
    mesh = plsc.VectorSubcoreMesh(core_axis_name="c", subcore_axis_name="s")

    @functools.partial(
        pl.kernel,
        out_type=jax.ShapeDtypeStruct((E, P), jnp.float32),
        mesh=mesh,
        scratch_types=[
            pltpu.VMEM((GC,), jnp.int32),
            pltpu.VMEM((GC,), jnp.int32),
            pltpu.VMEM((GC, P), jnp.float32),
            pltpu.SemaphoreType.DMA,
        ],
    )
    def k(tab_r_hbm, tab_c_hbm, row_hbm, col_hbm, out_hbm, ridx, cidx, buf, sem):
        wid = lax.axis_index("s") * NC + lax.axis_index("c")
        base = wid * PER_W

        def body(i, carry):
            off = base + i * GC
            pltpu.sync_copy(row_hbm.at[pl.ds(off, GC)], ridx)
            pltpu.sync_copy(col_hbm.at[pl.ds(off, GC)], cidx)
            pltpu.async_copy(tab_r_hbm.at[ridx], buf, sem).wait()
            pltpu.async_copy(tab_c_hbm.at[cidx], buf, sem, add=True).wait()
            pltpu.sync_copy(buf, out_hbm.at[pl.ds(off, GC)])
            return carry

        lax.fori_loop(0, PER_W // GC, body, 0)

    return k(tab_r, tab_c, row, col)


def _gather_one(tab, idx, nout, gc):
    P = tab.shape[1]
    perw = nout // NW
    mesh = plsc.VectorSubcoreMesh(core_axis_name="c", subcore_axis_name="s")

    @functools.partial(
        pl.kernel,
        out_type=jax.ShapeDtypeStruct((nout, P), jnp.float32),
        mesh=mesh,
        scratch_types=[
            pltpu.VMEM((gc,), jnp.int32),
            pltpu.VMEM((gc, P), jnp.float32),
            pltpu.SemaphoreType.DMA,
        ],
    )
    def k(tab_hbm, idx_hbm, out_hbm, ridx, buf, sem):
        wid = lax.axis_index("s") * NC + lax.axis_index("c")
        base = wid * perw

        def body(i, carry):
            off = base + i * gc
            pltpu.sync_copy(idx_hbm.at[pl.ds(off, gc)], ridx)
            pltpu.async_copy(tab_hbm.at[ridx], buf, sem).wait()
            pltpu.sync_copy(buf, out_hbm.at[pl.ds(off, gc)])
            return carry

        lax.fori_loop(0, perw // gc, body, 0)

    return k(tab, idx)


CSZ = 5000
N32 = 51200
ACCW = N32 * 32
SSTRIPE = ACCW // NS


def _sc_scatter(mtflat, eflat, jobs0, jobs1, nslots):
    mesh = plsc.VectorSubcoreMesh(core_axis_name="c", subcore_axis_name="s")

    @functools.partial(
        pl.kernel,
        out_type=jax.ShapeDtypeStruct((nslots, NS, SSTRIPE), jnp.float32),
        mesh=mesh,
        scratch_types=[
            pltpu.VMEM((CSZ,), jnp.int32),
            pltpu.VMEM((CSZ,), jnp.float32),
            pltpu.VMEM_SHARED((ACCW,), jnp.float32),
            pltpu.SemaphoreType.DMA,
        ],
    )
    def k(mt_hbm, ef_hbm, z_hbm, out_hbm, ibuf, dbuf, acc, sem):
        cc = lax.axis_index("c")
        s = lax.axis_index("s")

        def run_jobs(jobs):
            for (q, lo, hi, slot) in jobs:
                ept = (hi - lo) // NS
                pltpu.sync_copy(z_hbm.at[s], acc.at[pl.ds(s * SSTRIPE, SSTRIPE)])
                plsc.subcore_barrier()

                def jbody(j, carry):
                    def body(i, carry2):
                        eo = lo + s * ept + i * CSZ
                        pltpu.sync_copy(ef_hbm.at[pl.ds(j * E + eo, CSZ)], ibuf)
                        pltpu.sync_copy(
                            mt_hbm.at[pl.ds((q * 32 + j) * E + eo, CSZ)], dbuf)
                        pltpu.async_copy(dbuf, acc.at[ibuf], sem, add=True).wait()
                        return carry2

                    lax.fori_loop(0, ept // CSZ, body, 0)
                    return carry

                lax.fori_loop(0, 32, jbody, 0)
                plsc.subcore_barrier()
                pltpu.sync_copy(acc.at[pl.ds(s * SSTRIPE, SSTRIPE)],
                                out_hbm.at[slot, s])
                plsc.subcore_barrier()

        @pl.when(cc == 0)
        def _():
            run_jobs(jobs0)

        @pl.when(cc == 1)
        def _():
            run_jobs(jobs1)

    return k(mtflat, eflat, jnp.zeros((NS, SSTRIPE), jnp.float32))


def _make_eflat(row):
    ef = row[None, :] * 32 + jnp.arange(32, dtype=jnp.int32)[:, None]
    return ef.reshape(-1)


def _combine_slots(out, hid):
    a = out.reshape(out.shape[0], N32, 32)[:, :N, :]
    if hid == 16:
        agg = a[0] + a[1]
    elif hid == 32:
        agg = jnp.concatenate([a[0], a[1]], axis=1)
    else:
        agg = jnp.concatenate([a[0], a[1], a[2] + a[3]], axis=1)
    magg = agg[:, :hid]
    trans = agg[:, hid:hid + 3]
    cnt = agg[:, hid + 3]
    return magg, trans, cnt


def _edge_kernel(hid, WB, g_ref, ea_ref, wre_ref,
                 We2_ref, be2_ref, Wc1_ref, bc1_ref, Wc2_ref,
                 mt_ref):
    g = g_ref[...]
    pre = g[:, :hid]
    cd = g[:, hid:hid + 3]
    radial = jnp.sum(cd * cd, axis=1, keepdims=True)
    ea = ea_ref[...]
    rad_ea = jnp.concatenate([radial, ea], axis=1)
    pre = pre + jnp.dot(rad_ea, wre_ref[...], preferred_element_type=jnp.float32)
    m = _silu(pre)
    m = _silu(jnp.dot(m, We2_ref[...], preferred_element_type=jnp.float32)
              + be2_ref[...])
    tt = _silu(jnp.dot(m, Wc1_ref[...], preferred_element_type=jnp.float32)
               + bc1_ref[...])
    t = jnp.dot(tt, Wc2_ref[...], preferred_element_type=jnp.float32)
    ones = jnp.ones_like(t)
    pad = jnp.zeros((m.shape[0], WB - hid - 4), jnp.float32)
    mt_ref[...] = jnp.concatenate([m, cd * t, ones, pad], axis=1)


def _edge_mlp(g, edge_attr, p, WB):
    hid = p['We2'].shape[0]
    inf = (p['We1'].shape[0] - 2) // 2
    wre = p['We1'][2 * inf:]
    P = g.shape[1]
    grid = (E // TE,)
    erow = lambda i: (i, 0)
    wfull = lambda i: (0, 0)
    out = pl.pallas_call(
        functools.partial(_edge_kernel, hid, WB),
        grid=grid,
        in_specs=[
            pl.BlockSpec((TE, P), erow),
            pl.BlockSpec((TE, 1), erow),
            pl.BlockSpec(wre.shape, wfull),
            pl.BlockSpec(p['We2'].shape, wfull),
            pl.BlockSpec((1, hid), wfull),
            pl.BlockSpec(p['Wc1'].shape, wfull),
            pl.BlockSpec((1, hid), wfull),
            pl.BlockSpec(p['Wc2'].shape, wfull),
        ],
        out_specs=[
            pl.BlockSpec((TE, WB), lambda i: (i, 0)),
        ],
        out_shape=[
            jax.ShapeDtypeStruct((E, WB), jnp.float32),
        ],
        interpret=_INTERPRET,
    )(g, edge_attr,
      wre, p['We2'], p['be2'][None, :], p['Wc1'], p['bc1'][None, :], p['Wc2'])
    return out[0]


def _segment_sum(data, seg, num):
    return jax.ops.segment_sum(data, seg, num_segments=num)


def kernel(pos, edge_attr, params, edge_index, face, vertex2face, batch, ptr,
           face_len, vertex2face_len):
    row, col = edge_index[0], edge_index[1]

    centroid = jnp.mean(pos, axis=0, keepdims=True)
    p = pos - centroid
    mx = jnp.max(jnp.sqrt(jnp.sum(p ** 2, axis=1)))
    p = p / mx

    F3 = face.shape[1] * 3
    F3P = 307200
    ptab = jnp.pad(p, ((0, 0), (0, 125)))
    fidx_flat = jnp.concatenate(
        [face.reshape(-1), jnp.arange(F3P - F3, dtype=jnp.int32) % N])
    gp = _gather_one(ptab, fidx_flat, F3P, 960)
    F = face.shape[1]
    v0 = gp[0:F, :3]
    v1 = gp[F:2 * F, :3]
    v2 = gp[2 * F:3 * F, :3]
    fn = jnp.cross(v1 - v0, v2 - v0)
    face_area = jnp.sqrt(jnp.sum(fn ** 2, axis=1)) / 2.0
    vtx = vertex2face[:, 0]
    fidx = vertex2face[:, 1]
    aval = face_area[fidx]
    apack = jnp.stack([aval, jnp.ones_like(aval)], axis=1)
    agg2 = _segment_sum(apack, vtx, N)
    area = agg2[:, 0] / jnp.maximum(agg2[:, 1], 1.0)
    x = area[:, None] * params['feat_W'][0][None, :] + params['feat_b'][None, :]

    coord = p
    for lp in (params['c1'], params['c2'], params['c3']):
        hid = lp['We2'].shape[0]
        inf = (lp['We1'].shape[0] - 2) // 2
        A = lp['We1'][:inf]
        B = lp['We1'][inf:2 * inf]
        P = 128
        pad = jnp.zeros((N, P - hid - 3), jnp.float32)
        tab_r = jnp.concatenate([x @ A + lp['be1'][None, :], coord, pad], axis=1)
        tab_c = jnp.concatenate([x @ B, -coord, pad], axis=1)
        g = _gather_add(tab_r, tab_c, row, col)
        WB = {16: 24, 32: 40, 64: 72}[hid]
        mt = _edge_mlp(g, edge_attr, lp, WB)
        agg = _segment_sum(mt, row, N)
        magg = agg[:, :hid]
        trans = agg[:, hid:hid + 3]
        cnt = jnp.maximum(agg[:, hid + 3], 1.0)
        coord = coord + trans / cnt[:, None]
        h = jnp.concatenate([x, magg], axis=1)
        h = _silu(h @ lp['Wn1'] + lp['bn1'])
        x = h @ lp['Wn2'] + lp['bn2']

    x = jax.nn.relu(x @ params['lin1_W'] + params['lin1_b'])
    x = jnp.mean(x, axis=0, keepdims=True)
    x = x @ params['lin2_W'] + params['lin2_b']
    return jax.nn.log_softmax(x, axis=1)


# speedup vs baseline: 1.8136x; 1.0235x over previous
import functools

import jax
import jax.numpy as jnp
from jax import lax
from jax.experimental import pallas as pl
from jax.experimental.pallas import tpu as pltpu
from jax.experimental.pallas import tpu_sc as plsc

N = 50000
E = 800000

_INTERPRET = False

TE = 1280
NC, NS = 2, 16
NW = NC * NS
PER_W = E // NW
GC = 1000


def _silu(x):
    return x * jax.nn.sigmoid(x)


def _gather_add(tab_r, tab_c, row, col):
    P = tab_r.shape[1]
    mesh = plsc.VectorSubcoreMesh(core_axis_name="c", subcore_axis_name="s")

    @functools.partial(
        pl.kernel,
        out_type=jax.ShapeDtypeStruct((E, P), jnp.float32),
        mesh=mesh,
        scratch_types=[
            pltpu.VMEM((GC,), jnp.int32),
            pltpu.VMEM((GC,), jnp.int32),
            pltpu.VMEM((GC, P), jnp.float32),
            pltpu.SemaphoreType.DMA,
        ],
    )
    def k(tab_r_hbm, tab_c_hbm, row_hbm, col_hbm, out_hbm, ridx, cidx, buf, sem):
        wid = lax.axis_index("s") * NC + lax.axis_index("c")
        base = wid * PER_W

        def body(i, carry):
            off = base + i * GC
            pltpu.sync_copy(row_hbm.at[pl.ds(off, GC)], ridx)
            pltpu.sync_copy(col_hbm.at[pl.ds(off, GC)], cidx)
            pltpu.async_copy(tab_r_hbm.at[ridx], buf, sem).wait()
            pltpu.async_copy(tab_c_hbm.at[cidx], buf, sem, add=True).wait()
            pltpu.sync_copy(buf, out_hbm.at[pl.ds(off, GC)])
            return carry

        lax.fori_loop(0, PER_W // GC, body, 0)

    return k(tab_r, tab_c, row, col)


def _gather_one(tab, idx, nout, gc):
    P = tab.shape[1]
    perw = nout // NW
    mesh = plsc.VectorSubcoreMesh(core_axis_name="c", subcore_axis_name="s")

    @functools.partial(
        pl.kernel,
        out_type=jax.ShapeDtypeStruct((nout, P), jnp.float32),
        mesh=mesh,
        scratch_types=[
            pltpu.VMEM((gc,), jnp.int32),
            pltpu.VMEM((gc, P), jnp.float32),
            pltpu.SemaphoreType.DMA,
        ],
    )
    def k(tab_hbm, idx_hbm, out_hbm, ridx, buf, sem):
        wid = lax.axis_index("s") * NC + lax.axis_index("c")
        base = wid * perw

        def body(i, carry):
            off = base + i * gc
            pltpu.sync_copy(idx_hbm.at[pl.ds(off, gc)], ridx)
            pltpu.async_copy(tab_hbm.at[ridx], buf, sem).wait()
            pltpu.sync_copy(buf, out_hbm.at[pl.ds(off, gc)])
            return carry

        lax.fori_loop(0, perw // gc, body, 0)

    return k(tab, idx)


CSZ = 5000
N32 = 51200
ACCW = N32 * 32
SSTRIPE = ACCW // NS


def _sc_scatter(mtflat, eflat, jobs0, jobs1, nslots):
    mesh = plsc.VectorSubcoreMesh(core_axis_name="c", subcore_axis_name="s")

    @functools.partial(
        pl.kernel,
        out_type=jax.ShapeDtypeStruct((nslots, NS, SSTRIPE), jnp.float32),
        mesh=mesh,
        scratch_types=[
            pltpu.VMEM((CSZ,), jnp.int32),
            pltpu.VMEM((CSZ,), jnp.float32),
            pltpu.VMEM_SHARED((ACCW,), jnp.float32),
            pltpu.SemaphoreType.DMA,
        ],
    )
    def k(mt_hbm, ef_hbm, z_hbm, out_hbm, ibuf, dbuf, acc, sem):
        cc = lax.axis_index("c")
        s = lax.axis_index("s")

        def run_jobs(jobs):
            for (q, lo, hi, slot) in jobs:
                ept = (hi - lo) // NS
                pltpu.sync_copy(z_hbm.at[s], acc.at[pl.ds(s * SSTRIPE, SSTRIPE)])
                plsc.subcore_barrier()

                def jbody(j, carry):
                    def body(i, carry2):
                        eo = lo + s * ept + i * CSZ
                        pltpu.sync_copy(ef_hbm.at[pl.ds(j * E + eo, CSZ)], ibuf)
                        pltpu.sync_copy(
                            mt_hbm.at[pl.ds((q * 32 + j) * E + eo, CSZ)], dbuf)
                        pltpu.async_copy(dbuf, acc.at[ibuf], sem, add=True).wait()
                        return carry2

                    lax.fori_loop(0, ept // CSZ, body, 0)
                    return carry

                lax.fori_loop(0, 32, jbody, 0)
                plsc.subcore_barrier()
                pltpu.sync_copy(acc.at[pl.ds(s * SSTRIPE, SSTRIPE)],
                                out_hbm.at[slot, s])
                plsc.subcore_barrier()

        @pl.when(cc == 0)
        def _():
            run_jobs(jobs0)

        @pl.when(cc == 1)
        def _():
            run_jobs(jobs1)

    return k(mtflat, eflat, jnp.zeros((NS, SSTRIPE), jnp.float32))


def _make_eflat(row):
    ef = row[None, :] * 32 + jnp.arange(32, dtype=jnp.int32)[:, None]
    return ef.reshape(-1)


def _combine_slots(out, hid):
    a = out.reshape(out.shape[0], N32, 32)[:, :N, :]
    if hid == 16:
        agg = a[0] + a[1]
    elif hid == 32:
        agg = jnp.concatenate([a[0], a[1]], axis=1)
    else:
        agg = jnp.concatenate([a[0], a[1], a[2] + a[3]], axis=1)
    magg = agg[:, :hid]
    trans = agg[:, hid:hid + 3]
    cnt = agg[:, hid + 3]
    return magg, trans, cnt


def _edge_kernel(hid, WB, g_ref, ea_ref, wre_ref,
                 We2_ref, be2_ref, Wc1_ref, bc1_ref, Wc2_ref,
                 mt_ref):
    g = g_ref[...]
    pre = g[:, :hid]
    cd = g[:, hid:hid + 3]
    radial = jnp.sum(cd * cd, axis=1, keepdims=True)
    ea = ea_ref[...]
    rad_ea = jnp.concatenate([radial, ea], axis=1)
    pre = pre + jnp.dot(rad_ea, wre_ref[...], preferred_element_type=jnp.float32)
    m = _silu(pre)
    m = _silu(jnp.dot(m, We2_ref[...], preferred_element_type=jnp.float32)
              + be2_ref[...])
    tt = _silu(jnp.dot(m, Wc1_ref[...], preferred_element_type=jnp.float32)
               + bc1_ref[...])
    t = jnp.dot(tt, Wc2_ref[...], preferred_element_type=jnp.float32)
    ones = jnp.ones_like(t)
    pad = jnp.zeros((m.shape[0], WB - hid - 4), jnp.float32)
    mt_ref[...] = jnp.concatenate([m, cd * t, ones, pad], axis=1)


def _edge_mlp(g, edge_attr, p, WB):
    hid = p['We2'].shape[0]
    inf = (p['We1'].shape[0] - 2) // 2
    wre = p['We1'][2 * inf:]
    P = g.shape[1]
    grid = (E // TE,)
    erow = lambda i: (i, 0)
    wfull = lambda i: (0, 0)
    out = pl.pallas_call(
        functools.partial(_edge_kernel, hid, WB),
        grid=grid,
        in_specs=[
            pl.BlockSpec((TE, P), erow),
            pl.BlockSpec((TE, 1), erow),
            pl.BlockSpec(wre.shape, wfull),
            pl.BlockSpec(p['We2'].shape, wfull),
            pl.BlockSpec((1, hid), wfull),
            pl.BlockSpec(p['Wc1'].shape, wfull),
            pl.BlockSpec((1, hid), wfull),
            pl.BlockSpec(p['Wc2'].shape, wfull),
        ],
        out_specs=[
            pl.BlockSpec((TE, WB), lambda i: (i, 0)),
        ],
        out_shape=[
            jax.ShapeDtypeStruct((E, WB), jnp.float32),
        ],
        interpret=_INTERPRET,
    )(g, edge_attr,
      wre, p['We2'], p['be2'][None, :], p['Wc1'], p['bc1'][None, :], p['Wc2'])
    return out[0]


def _segment_sum(data, seg, num):
    return jax.ops.segment_sum(data, seg, num_segments=num)


def kernel(pos, edge_attr, params, edge_index, face, vertex2face, batch, ptr,
           face_len, vertex2face_len):
    row, col = edge_index[0], edge_index[1]

    centroid = jnp.mean(pos, axis=0, keepdims=True)
    p = pos - centroid
    mx = jnp.max(jnp.sqrt(jnp.sum(p ** 2, axis=1)))
    p = p / mx

    F3 = face.shape[1] * 3
    F3P = 307200
    ptab = jnp.pad(p, ((0, 0), (0, 125)))
    fidx_flat = jnp.concatenate(
        [face.reshape(-1), jnp.arange(F3P - F3, dtype=jnp.int32) % N])
    gp = _gather_one(ptab, fidx_flat, F3P, 960)
    F = face.shape[1]
    v0 = gp[0:F, :3]
    v1 = gp[F:2 * F, :3]
    v2 = gp[2 * F:3 * F, :3]
    fn = jnp.cross(v1 - v0, v2 - v0)
    face_area = jnp.sqrt(jnp.sum(fn ** 2, axis=1)) / 2.0
    vtx = vertex2face[:, 0]
    fidx = vertex2face[:, 1]
    asum = _segment_sum(face_area[fidx], vtx, N)
    acnt = jnp.maximum(_segment_sum(jnp.ones((vtx.shape[0],), jnp.float32), vtx, N), 1.0)
    area = asum / acnt
    x = area[:, None] * params['feat_W'][0][None, :] + params['feat_b'][None, :]

    coord = p
    for lp in (params['c1'], params['c2'], params['c3']):
        hid = lp['We2'].shape[0]
        inf = (lp['We1'].shape[0] - 2) // 2
        A = lp['We1'][:inf]
        B = lp['We1'][inf:2 * inf]
        P = 128
        pad = jnp.zeros((N, P - hid - 3), jnp.float32)
        tab_r = jnp.concatenate([x @ A + lp['be1'][None, :], coord, pad], axis=1)
        tab_c = jnp.concatenate([x @ B, -coord, pad], axis=1)
        g = _gather_add(tab_r, tab_c, row, col)
        WB = {16: 24, 32: 40, 64: 72}[hid]
        mt = _edge_mlp(g, edge_attr, lp, WB)
        agg = _segment_sum(mt, row, N)
        magg = agg[:, :hid]
        trans = agg[:, hid:hid + 3]
        cnt = jnp.maximum(agg[:, hid + 3], 1.0)
        coord = coord + trans / cnt[:, None]
        h = jnp.concatenate([x, magg], axis=1)
        h = _silu(h @ lp['Wn1'] + lp['bn1'])
        x = h @ lp['Wn2'] + lp['bn2']

    x = jax.nn.relu(x @ params['lin1_W'] + params['lin1_b'])
    x = jnp.mean(x, axis=0, keepdims=True)
    x = x @ params['lin2_W'] + params['lin2_b']
    return jax.nn.log_softmax(x, axis=1)


# speedup vs baseline: 1.9700x; 1.0863x over previous
import functools

import jax
import jax.numpy as jnp
from jax import lax
from jax.experimental import pallas as pl
from jax.experimental.pallas import tpu as pltpu
from jax.experimental.pallas import tpu_sc as plsc

N = 50000
E = 800000

_INTERPRET = False

TE = 1280
NC, NS = 2, 16
NW = NC * NS
PER_W = E // NW
GC = 1000


def _silu(x):
    return x * jax.nn.sigmoid(x)


def _gather_add(tab_r, tab_c, row, col):
    P = tab_r.shape[1]
    mesh = plsc.VectorSubcoreMesh(core_axis_name="c", subcore_axis_name="s")

    @functools.partial(
        pl.kernel,
        out_type=jax.ShapeDtypeStruct((E, P), jnp.float32),
        mesh=mesh,
        scratch_types=[
            pltpu.VMEM((GC,), jnp.int32),
            pltpu.VMEM((GC,), jnp.int32),
            pltpu.VMEM((GC, P), jnp.float32),
            pltpu.SemaphoreType.DMA,
        ],
    )
    def k(tab_r_hbm, tab_c_hbm, row_hbm, col_hbm, out_hbm, ridx, cidx, buf, sem):
        wid = lax.axis_index("s") * NC + lax.axis_index("c")
        base = wid * PER_W

        def body(i, carry):
            off = base + i * GC
            pltpu.sync_copy(row_hbm.at[pl.ds(off, GC)], ridx)
            pltpu.sync_copy(col_hbm.at[pl.ds(off, GC)], cidx)
            pltpu.async_copy(tab_r_hbm.at[ridx], buf, sem).wait()
            pltpu.async_copy(tab_c_hbm.at[cidx], buf, sem, add=True).wait()
            pltpu.sync_copy(buf, out_hbm.at[pl.ds(off, GC)])
            return carry

        lax.fori_loop(0, PER_W // GC, body, 0)

    return k(tab_r, tab_c, row, col)


def _gather_one(tab, idx, nout, gc):
    P = tab.shape[1]
    perw = nout // NW
    mesh = plsc.VectorSubcoreMesh(core_axis_name="c", subcore_axis_name="s")

    @functools.partial(
        pl.kernel,
        out_type=jax.ShapeDtypeStruct((nout, P), jnp.float32),
        mesh=mesh,
        scratch_types=[
            pltpu.VMEM((gc,), jnp.int32),
            pltpu.VMEM((gc, P), jnp.float32),
            pltpu.SemaphoreType.DMA,
        ],
    )
    def k(tab_hbm, idx_hbm, out_hbm, ridx, buf, sem):
        wid = lax.axis_index("s") * NC + lax.axis_index("c")
        base = wid * perw

        def body(i, carry):
            off = base + i * gc
            pltpu.sync_copy(idx_hbm.at[pl.ds(off, gc)], ridx)
            pltpu.async_copy(tab_hbm.at[ridx], buf, sem).wait()
            pltpu.sync_copy(buf, out_hbm.at[pl.ds(off, gc)])
            return carry

        lax.fori_loop(0, perw // gc, body, 0)

    return k(tab, idx)


CSZ = 5000
N32 = 51200
ACCW = N32 * 32
SSTRIPE = ACCW // NS


def _sc_scatter(mtflat, eflat, jobs0, jobs1, nslots):
    mesh = plsc.VectorSubcoreMesh(core_axis_name="c", subcore_axis_name="s")

    @functools.partial(
        pl.kernel,
        out_type=jax.ShapeDtypeStruct((nslots, NS, SSTRIPE), jnp.float32),
        mesh=mesh,
        scratch_types=[
            pltpu.VMEM((CSZ,), jnp.int32),
            pltpu.VMEM((CSZ,), jnp.float32),
            pltpu.VMEM_SHARED((ACCW,), jnp.float32),
            pltpu.SemaphoreType.DMA,
        ],
    )
    def k(mt_hbm, ef_hbm, z_hbm, out_hbm, ibuf, dbuf, acc, sem):
        cc = lax.axis_index("c")
        s = lax.axis_index("s")

        def run_jobs(jobs):
            for (q, lo, hi, slot) in jobs:
                ept = (hi - lo) // NS
                pltpu.sync_copy(z_hbm.at[s], acc.at[pl.ds(s * SSTRIPE, SSTRIPE)])
                plsc.subcore_barrier()

                def jbody(j, carry):
                    def body(i, carry2):
                        eo = lo + s * ept + i * CSZ
                        pltpu.sync_copy(ef_hbm.at[pl.ds(j * E + eo, CSZ)], ibuf)
                        pltpu.sync_copy(
                            mt_hbm.at[pl.ds((q * 32 + j) * E + eo, CSZ)], dbuf)
                        pltpu.async_copy(dbuf, acc.at[ibuf], sem, add=True).wait()
                        return carry2

                    lax.fori_loop(0, ept // CSZ, body, 0)
                    return carry

                lax.fori_loop(0, 32, jbody, 0)
                plsc.subcore_barrier()
                pltpu.sync_copy(acc.at[pl.ds(s * SSTRIPE, SSTRIPE)],
                                out_hbm.at[slot, s])
                plsc.subcore_barrier()

        @pl.when(cc == 0)
        def _():
            run_jobs(jobs0)

        @pl.when(cc == 1)
        def _():
            run_jobs(jobs1)

    return k(mtflat, eflat, jnp.zeros((NS, SSTRIPE), jnp.float32))


def _make_eflat(row):
    ef = row[None, :] * 32 + jnp.arange(32, dtype=jnp.int32)[:, None]
    return ef.reshape(-1)


def _combine_slots(out, hid):
    a = out.reshape(out.shape[0], N32, 32)[:, :N, :]
    if hid == 16:
        agg = a[0] + a[1]
    elif hid == 32:
        agg = jnp.concatenate([a[0], a[1]], axis=1)
    else:
        agg = jnp.concatenate([a[0], a[1], a[2] + a[3]], axis=1)
    magg = agg[:, :hid]
    trans = agg[:, hid:hid + 3]
    cnt = agg[:, hid + 3]
    return magg, trans, cnt


def _edge_kernel(hid, WB, g_ref, ea_ref, wre_ref,
                 We2_ref, be2_ref, Wc1_ref, bc1_ref, Wc2_ref,
                 mt_ref):
    g = g_ref[...]
    pre = g[:, :hid]
    cd = g[:, hid:hid + 3]
    radial = jnp.sum(cd * cd, axis=1, keepdims=True)
    ea = ea_ref[...]
    rad_ea = jnp.concatenate([radial, ea], axis=1)
    pre = pre + jnp.dot(rad_ea, wre_ref[...], preferred_element_type=jnp.float32)
    m = _silu(pre)
    m = _silu(jnp.dot(m, We2_ref[...], preferred_element_type=jnp.float32)
              + be2_ref[...])
    tt = _silu(jnp.dot(m, Wc1_ref[...], preferred_element_type=jnp.float32)
               + bc1_ref[...])
    t = jnp.dot(tt, Wc2_ref[...], preferred_element_type=jnp.float32)
    if WB == hid + 4:
        ones = jnp.ones_like(t)
        mt_ref[...] = jnp.concatenate([m, cd * t, ones], axis=1)
    else:
        mt_ref[...] = jnp.concatenate([m, cd * t], axis=1)


def _edge_mlp(g, edge_attr, p, WB):
    hid = p['We2'].shape[0]
    inf = (p['We1'].shape[0] - 2) // 2
    wre = p['We1'][2 * inf:]
    P = g.shape[1]
    grid = (E // TE,)
    erow = lambda i: (i, 0)
    wfull = lambda i: (0, 0)
    out = pl.pallas_call(
        functools.partial(_edge_kernel, hid, WB),
        grid=grid,
        in_specs=[
            pl.BlockSpec((TE, P), erow),
            pl.BlockSpec((TE, 1), erow),
            pl.BlockSpec(wre.shape, wfull),
            pl.BlockSpec(p['We2'].shape, wfull),
            pl.BlockSpec((1, hid), wfull),
            pl.BlockSpec(p['Wc1'].shape, wfull),
            pl.BlockSpec((1, hid), wfull),
            pl.BlockSpec(p['Wc2'].shape, wfull),
        ],
        out_specs=[
            pl.BlockSpec((TE, WB), lambda i: (i, 0)),
        ],
        out_shape=[
            jax.ShapeDtypeStruct((E, WB), jnp.float32),
        ],
        interpret=_INTERPRET,
    )(g, edge_attr,
      wre, p['We2'], p['be2'][None, :], p['Wc1'], p['bc1'][None, :], p['Wc2'])
    return out[0]


def _segment_sum(data, seg, num):
    return jax.ops.segment_sum(data, seg, num_segments=num)


def kernel(pos, edge_attr, params, edge_index, face, vertex2face, batch, ptr,
           face_len, vertex2face_len):
    row, col = edge_index[0], edge_index[1]

    centroid = jnp.mean(pos, axis=0, keepdims=True)
    p = pos - centroid
    mx = jnp.max(jnp.sqrt(jnp.sum(p ** 2, axis=1)))
    p = p / mx

    v0 = p[face[0]]
    v1 = p[face[1]]
    v2 = p[face[2]]
    fn = jnp.cross(v1 - v0, v2 - v0)
    face_area = jnp.sqrt(jnp.sum(fn ** 2, axis=1)) / 2.0
    vtx = vertex2face[:, 0]
    aval = jnp.concatenate([face_area, face_area, face_area])
    asum = _segment_sum(aval, vtx, N)
    acnt = jnp.maximum(_segment_sum(jnp.ones((vtx.shape[0],), jnp.float32), vtx, N), 1.0)
    area = asum / acnt
    x = area[:, None] * params['feat_W'][0][None, :] + params['feat_b'][None, :]

    coord = p
    for lp in (params['c1'], params['c2'], params['c3']):
        hid = lp['We2'].shape[0]
        inf = (lp['We1'].shape[0] - 2) // 2
        A = lp['We1'][:inf]
        B = lp['We1'][inf:2 * inf]
        P = 128
        pad = jnp.zeros((N, P - hid - 3), jnp.float32)
        tab_r = jnp.concatenate([x @ A + lp['be1'][None, :], coord, pad], axis=1)
        tab_c = jnp.concatenate([x @ B, -coord, pad], axis=1)
        g = _gather_add(tab_r, tab_c, row, col)
        first = lp is params['c1']
        WB = hid + 4 if first else hid + 3
        mt = _edge_mlp(g, edge_attr, lp, WB)
        agg = _segment_sum(mt, row, N)
        magg = agg[:, :hid]
        trans = agg[:, hid:hid + 3]
        if first:
            cnt = jnp.maximum(agg[:, hid + 3], 1.0)
        coord = coord + trans / cnt[:, None]
        h = jnp.concatenate([x, magg], axis=1)
        h = _silu(h @ lp['Wn1'] + lp['bn1'])
        x = h @ lp['Wn2'] + lp['bn2']

    x = jax.nn.relu(x @ params['lin1_W'] + params['lin1_b'])
    x = jnp.mean(x, axis=0, keepdims=True)
    x = x @ params['lin2_W'] + params['lin2_b']
    return jax.nn.log_softmax(x, axis=1)


# speedup vs baseline: 2.0330x; 1.0320x over previous
import functools

import jax
import jax.numpy as jnp
from jax import lax
from jax.experimental import pallas as pl
from jax.experimental.pallas import tpu as pltpu
from jax.experimental.pallas import tpu_sc as plsc

N = 50000
E = 800000

_INTERPRET = False

TE = 1280
NC, NS = 2, 16
NW = NC * NS
PER_W = E // NW
GC = 1000


def _silu(x):
    return x * jax.nn.sigmoid(x)


def _gather_add(tab_r, tab_c, row, col):
    P = tab_r.shape[1]
    mesh = plsc.VectorSubcoreMesh(core_axis_name="c", subcore_axis_name="s")

    @functools.partial(
        pl.kernel,
        out_type=jax.ShapeDtypeStruct((E, P), jnp.float32),
        mesh=mesh,
        scratch_types=[
            pltpu.VMEM((GC,), jnp.int32),
            pltpu.VMEM((GC,), jnp.int32),
            pltpu.VMEM((GC, P), jnp.float32),
            pltpu.SemaphoreType.DMA,
        ],
    )
    def k(tab_r_hbm, tab_c_hbm, row_hbm, col_hbm, out_hbm, ridx, cidx, buf, sem):
        wid = lax.axis_index("s") * NC + lax.axis_index("c")
        base = wid * PER_W

        def body(i, carry):
            off = base + i * GC
            pltpu.sync_copy(row_hbm.at[pl.ds(off, GC)], ridx)
            pltpu.sync_copy(col_hbm.at[pl.ds(off, GC)], cidx)
            pltpu.async_copy(tab_r_hbm.at[ridx], buf, sem).wait()
            pltpu.async_copy(tab_c_hbm.at[cidx], buf, sem, add=True).wait()
            pltpu.sync_copy(buf, out_hbm.at[pl.ds(off, GC)])
            return carry

        lax.fori_loop(0, PER_W // GC, body, 0)

    return k(tab_r, tab_c, row, col)


def _gather_one(tab, idx, nout, gc):
    P = tab.shape[1]
    perw = nout // NW
    mesh = plsc.VectorSubcoreMesh(core_axis_name="c", subcore_axis_name="s")

    @functools.partial(
        pl.kernel,
        out_type=jax.ShapeDtypeStruct((nout, P), jnp.float32),
        mesh=mesh,
        scratch_types=[
            pltpu.VMEM((gc,), jnp.int32),
            pltpu.VMEM((gc, P), jnp.float32),
            pltpu.SemaphoreType.DMA,
        ],
    )
    def k(tab_hbm, idx_hbm, out_hbm, ridx, buf, sem):
        wid = lax.axis_index("s") * NC + lax.axis_index("c")
        base = wid * perw

        def body(i, carry):
            off = base + i * gc
            pltpu.sync_copy(idx_hbm.at[pl.ds(off, gc)], ridx)
            pltpu.async_copy(tab_hbm.at[ridx], buf, sem).wait()
            pltpu.sync_copy(buf, out_hbm.at[pl.ds(off, gc)])
            return carry

        lax.fori_loop(0, perw // gc, body, 0)

    return k(tab, idx)


CSZ = 5000
N32 = 51200
ACCW = N32 * 32
SSTRIPE = ACCW // NS


def _sc_scatter(mtflat, eflat, jobs0, jobs1, nslots):
    mesh = plsc.VectorSubcoreMesh(core_axis_name="c", subcore_axis_name="s")

    @functools.partial(
        pl.kernel,
        out_type=jax.ShapeDtypeStruct((nslots, NS, SSTRIPE), jnp.float32),
        mesh=mesh,
        scratch_types=[
            pltpu.VMEM((CSZ,), jnp.int32),
            pltpu.VMEM((CSZ,), jnp.float32),
            pltpu.VMEM_SHARED((ACCW,), jnp.float32),
            pltpu.SemaphoreType.DMA,
        ],
    )
    def k(mt_hbm, ef_hbm, z_hbm, out_hbm, ibuf, dbuf, acc, sem):
        cc = lax.axis_index("c")
        s = lax.axis_index("s")

        def run_jobs(jobs):
            for (q, lo, hi, slot) in jobs:
                ept = (hi - lo) // NS
                pltpu.sync_copy(z_hbm.at[s], acc.at[pl.ds(s * SSTRIPE, SSTRIPE)])
                plsc.subcore_barrier()

                def jbody(j, carry):
                    def body(i, carry2):
                        eo = lo + s * ept + i * CSZ
                        pltpu.sync_copy(ef_hbm.at[pl.ds(j * E + eo, CSZ)], ibuf)
                        pltpu.sync_copy(
                            mt_hbm.at[pl.ds((q * 32 + j) * E + eo, CSZ)], dbuf)
                        pltpu.async_copy(dbuf, acc.at[ibuf], sem, add=True).wait()
                        return carry2

                    lax.fori_loop(0, ept // CSZ, body, 0)
                    return carry

                lax.fori_loop(0, 32, jbody, 0)
                plsc.subcore_barrier()
                pltpu.sync_copy(acc.at[pl.ds(s * SSTRIPE, SSTRIPE)],
                                out_hbm.at[slot, s])
                plsc.subcore_barrier()

        @pl.when(cc == 0)
        def _():
            run_jobs(jobs0)

        @pl.when(cc == 1)
        def _():
            run_jobs(jobs1)

    return k(mtflat, eflat, jnp.zeros((NS, SSTRIPE), jnp.float32))


def _make_eflat(row):
    ef = row[None, :] * 32 + jnp.arange(32, dtype=jnp.int32)[:, None]
    return ef.reshape(-1)


def _combine_slots(out, hid):
    a = out.reshape(out.shape[0], N32, 32)[:, :N, :]
    if hid == 16:
        agg = a[0] + a[1]
    elif hid == 32:
        agg = jnp.concatenate([a[0], a[1]], axis=1)
    else:
        agg = jnp.concatenate([a[0], a[1], a[2] + a[3]], axis=1)
    magg = agg[:, :hid]
    trans = agg[:, hid:hid + 3]
    cnt = agg[:, hid + 3]
    return magg, trans, cnt


def _edge_kernel(hid, WB, g_ref, ea_ref, wre_ref,
                 We2_ref, be2_ref, Wc1_ref, bc1_ref, Wc2_ref,
                 mt_ref):
    g = g_ref[...]
    pre = g[:, :hid]
    cd = g[:, hid:hid + 3]
    radial = jnp.sum(cd * cd, axis=1, keepdims=True)
    ea = ea_ref[...]
    rad_ea = jnp.concatenate([radial, ea], axis=1)
    pre = pre + jnp.dot(rad_ea, wre_ref[...], preferred_element_type=jnp.float32)
    m = _silu(pre)
    m = _silu(jnp.dot(m, We2_ref[...], preferred_element_type=jnp.float32)
              + be2_ref[...])
    tt = _silu(jnp.dot(m, Wc1_ref[...], preferred_element_type=jnp.float32)
               + bc1_ref[...])
    t = jnp.dot(tt, Wc2_ref[...], preferred_element_type=jnp.float32)
    if WB == hid + 4:
        ones = jnp.ones_like(t)
        mt_ref[...] = jnp.concatenate([m, cd * t, ones], axis=1)
    else:
        mt_ref[...] = jnp.concatenate([m, cd * t], axis=1)


def _edge_mlp(g, edge_attr, p, WB):
    hid = p['We2'].shape[0]
    inf = (p['We1'].shape[0] - 2) // 2
    wre = p['We1'][2 * inf:]
    P = g.shape[1]
    grid = (E // TE,)
    erow = lambda i: (i, 0)
    wfull = lambda i: (0, 0)
    out = pl.pallas_call(
        functools.partial(_edge_kernel, hid, WB),
        grid=grid,
        in_specs=[
            pl.BlockSpec((TE, P), erow),
            pl.BlockSpec((TE, 1), erow),
            pl.BlockSpec(wre.shape, wfull),
            pl.BlockSpec(p['We2'].shape, wfull),
            pl.BlockSpec((1, hid), wfull),
            pl.BlockSpec(p['Wc1'].shape, wfull),
            pl.BlockSpec((1, hid), wfull),
            pl.BlockSpec(p['Wc2'].shape, wfull),
        ],
        out_specs=[
            pl.BlockSpec((TE, WB), lambda i: (i, 0)),
        ],
        out_shape=[
            jax.ShapeDtypeStruct((E, WB), jnp.float32),
        ],
        interpret=_INTERPRET,
    )(g, edge_attr,
      wre, p['We2'], p['be2'][None, :], p['Wc1'], p['bc1'][None, :], p['Wc2'])
    return out[0]


def _segment_sum(data, seg, num):
    return jax.ops.segment_sum(data, seg, num_segments=num)


def kernel(pos, edge_attr, params, edge_index, face, vertex2face, batch, ptr,
           face_len, vertex2face_len):
    row, col = edge_index[0], edge_index[1]

    centroid = jnp.mean(pos, axis=0, keepdims=True)
    p = pos - centroid
    mx = jnp.max(jnp.sqrt(jnp.sum(p ** 2, axis=1)))
    p = p / mx

    v0 = p[face[0]]
    v1 = p[face[1]]
    v2 = p[face[2]]
    fn = jnp.cross(v1 - v0, v2 - v0)
    face_area = jnp.sqrt(jnp.sum(fn ** 2, axis=1)) / 2.0
    vtx = vertex2face[:, 0]
    aval = jnp.concatenate([face_area, face_area, face_area])
    asum = _segment_sum(aval, vtx, N)
    acnt = jnp.maximum(_segment_sum(jnp.ones((vtx.shape[0],), jnp.float32), vtx, N), 1.0)
    area = asum / acnt
    x = area[:, None] * params['feat_W'][0][None, :] + params['feat_b'][None, :]

    coord = p
    for lp in (params['c1'], params['c2'], params['c3']):
        hid = lp['We2'].shape[0]
        inf = (lp['We1'].shape[0] - 2) // 2
        A = lp['We1'][:inf]
        B = lp['We1'][inf:2 * inf]
        P = 128
        pad = jnp.zeros((N, P - hid - 3), jnp.float32)
        tab_r = jnp.concatenate([x @ A + lp['be1'][None, :], coord, pad], axis=1)
        tab_c = jnp.concatenate([x @ B, -coord, pad], axis=1)
        g = _gather_add(tab_r, tab_c, row, col)
        WB = hid + 4
        mt = _edge_mlp(g, edge_attr, lp, WB)
        agg = _segment_sum(mt, row, N)
        magg = agg[:, :hid]
        trans = agg[:, hid:hid + 3]
        cnt = jnp.maximum(agg[:, hid + 3], 1.0)
        coord = coord + trans / cnt[:, None]
        h = jnp.concatenate([x, magg], axis=1)
        h = _silu(h @ lp['Wn1'] + lp['bn1'])
        x = h @ lp['Wn2'] + lp['bn2']

    x = jax.nn.relu(x @ params['lin1_W'] + params['lin1_b'])
    x = jnp.mean(x, axis=0, keepdims=True)
    x = x @ params['lin2_W'] + params['lin2_b']
    return jax.nn.log_softmax(x, axis=1)


# speedup vs baseline: 2.1198x; 1.0427x over previous
import functools

import jax
import jax.numpy as jnp
from jax import lax
from jax.experimental import pallas as pl
from jax.experimental.pallas import tpu as pltpu
from jax.experimental.pallas import tpu_sc as plsc

N = 50000
E = 800000

_INTERPRET = False

TE = 1600
NC, NS = 2, 16
NW = NC * NS
PER_W = E // NW
GC = 1000


def _silu(x):
    return x * jax.nn.sigmoid(x)


def _gather_add(tab_r, tab_c, row, col):
    P = tab_r.shape[1]
    mesh = plsc.VectorSubcoreMesh(core_axis_name="c", subcore_axis_name="s")

    @functools.partial(
        pl.kernel,
        out_type=jax.ShapeDtypeStruct((E, P), jnp.float32),
        mesh=mesh,
        scratch_types=[
            pltpu.VMEM((GC,), jnp.int32),
            pltpu.VMEM((GC,), jnp.int32),
            pltpu.VMEM((GC, P), jnp.float32),
            pltpu.SemaphoreType.DMA,
        ],
    )
    def k(tab_r_hbm, tab_c_hbm, row_hbm, col_hbm, out_hbm, ridx, cidx, buf, sem):
        wid = lax.axis_index("s") * NC + lax.axis_index("c")
        base = wid * PER_W

        def body(i, carry):
            off = base + i * GC
            pltpu.sync_copy(row_hbm.at[pl.ds(off, GC)], ridx)
            pltpu.sync_copy(col_hbm.at[pl.ds(off, GC)], cidx)
            pltpu.async_copy(tab_r_hbm.at[ridx], buf, sem).wait()
            pltpu.async_copy(tab_c_hbm.at[cidx], buf, sem, add=True).wait()
            pltpu.sync_copy(buf, out_hbm.at[pl.ds(off, GC)])
            return carry

        lax.fori_loop(0, PER_W // GC, body, 0)

    return k(tab_r, tab_c, row, col)


def _gather_one(tab, idx, nout, gc):
    P = tab.shape[1]
    perw = nout // NW
    mesh = plsc.VectorSubcoreMesh(core_axis_name="c", subcore_axis_name="s")

    @functools.partial(
        pl.kernel,
        out_type=jax.ShapeDtypeStruct((nout, P), jnp.float32),
        mesh=mesh,
        scratch_types=[
            pltpu.VMEM((gc,), jnp.int32),
            pltpu.VMEM((gc, P), jnp.float32),
            pltpu.SemaphoreType.DMA,
        ],
    )
    def k(tab_hbm, idx_hbm, out_hbm, ridx, buf, sem):
        wid = lax.axis_index("s") * NC + lax.axis_index("c")
        base = wid * perw

        def body(i, carry):
            off = base + i * gc
            pltpu.sync_copy(idx_hbm.at[pl.ds(off, gc)], ridx)
            pltpu.async_copy(tab_hbm.at[ridx], buf, sem).wait()
            pltpu.sync_copy(buf, out_hbm.at[pl.ds(off, gc)])
            return carry

        lax.fori_loop(0, perw // gc, body, 0)

    return k(tab, idx)


CSZ = 5000
N32 = 51200
ACCW = N32 * 32
SSTRIPE = ACCW // NS


def _sc_scatter(mtflat, eflat, jobs0, jobs1, nslots):
    mesh = plsc.VectorSubcoreMesh(core_axis_name="c", subcore_axis_name="s")

    @functools.partial(
        pl.kernel,
        out_type=jax.ShapeDtypeStruct((nslots, NS, SSTRIPE), jnp.float32),
        mesh=mesh,
        scratch_types=[
            pltpu.VMEM((CSZ,), jnp.int32),
            pltpu.VMEM((CSZ,), jnp.float32),
            pltpu.VMEM_SHARED((ACCW,), jnp.float32),
            pltpu.SemaphoreType.DMA,
        ],
    )
    def k(mt_hbm, ef_hbm, z_hbm, out_hbm, ibuf, dbuf, acc, sem):
        cc = lax.axis_index("c")
        s = lax.axis_index("s")

        def run_jobs(jobs):
            for (q, lo, hi, slot) in jobs:
                ept = (hi - lo) // NS
                pltpu.sync_copy(z_hbm.at[s], acc.at[pl.ds(s * SSTRIPE, SSTRIPE)])
                plsc.subcore_barrier()

                def jbody(j, carry):
                    def body(i, carry2):
                        eo = lo + s * ept + i * CSZ
                        pltpu.sync_copy(ef_hbm.at[pl.ds(j * E + eo, CSZ)], ibuf)
                        pltpu.sync_copy(
                            mt_hbm.at[pl.ds((q * 32 + j) * E + eo, CSZ)], dbuf)
                        pltpu.async_copy(dbuf, acc.at[ibuf], sem, add=True).wait()
                        return carry2

                    lax.fori_loop(0, ept // CSZ, body, 0)
                    return carry

                lax.fori_loop(0, 32, jbody, 0)
                plsc.subcore_barrier()
                pltpu.sync_copy(acc.at[pl.ds(s * SSTRIPE, SSTRIPE)],
                                out_hbm.at[slot, s])
                plsc.subcore_barrier()

        @pl.when(cc == 0)
        def _():
            run_jobs(jobs0)

        @pl.when(cc == 1)
        def _():
            run_jobs(jobs1)

    return k(mtflat, eflat, jnp.zeros((NS, SSTRIPE), jnp.float32))


def _make_eflat(row):
    ef = row[None, :] * 32 + jnp.arange(32, dtype=jnp.int32)[:, None]
    return ef.reshape(-1)


def _combine_slots(out, hid):
    a = out.reshape(out.shape[0], N32, 32)[:, :N, :]
    if hid == 16:
        agg = a[0] + a[1]
    elif hid == 32:
        agg = jnp.concatenate([a[0], a[1]], axis=1)
    else:
        agg = jnp.concatenate([a[0], a[1], a[2] + a[3]], axis=1)
    magg = agg[:, :hid]
    trans = agg[:, hid:hid + 3]
    cnt = agg[:, hid + 3]
    return magg, trans, cnt


def _edge_kernel(hid, WB, g_ref, ea_ref, wre_ref,
                 We2_ref, be2_ref, Wc1_ref, bc1_ref, Wc2_ref,
                 mt_ref):
    g = g_ref[...]
    pre = g[:, :hid]
    cd = g[:, hid:hid + 3]
    radial = jnp.sum(cd * cd, axis=1, keepdims=True)
    ea = ea_ref[...]
    rad_ea = jnp.concatenate([radial, ea], axis=1)
    pre = pre + jnp.dot(rad_ea, wre_ref[...], preferred_element_type=jnp.float32)
    m = _silu(pre)
    m = _silu(jnp.dot(m, We2_ref[...], preferred_element_type=jnp.float32)
              + be2_ref[...])
    tt = _silu(jnp.dot(m, Wc1_ref[...], preferred_element_type=jnp.float32)
               + bc1_ref[...])
    t = jnp.dot(tt, Wc2_ref[...], preferred_element_type=jnp.float32)
    if WB == hid + 4:
        ones = jnp.ones_like(t)
        mt_ref[...] = jnp.concatenate([m, cd * t, ones], axis=1)
    else:
        mt_ref[...] = jnp.concatenate([m, cd * t], axis=1)


def _edge_mlp(g, edge_attr, p, WB):
    hid = p['We2'].shape[0]
    inf = (p['We1'].shape[0] - 2) // 2
    wre = p['We1'][2 * inf:]
    P = g.shape[1]
    grid = (E // TE,)
    erow = lambda i: (i, 0)
    wfull = lambda i: (0, 0)
    out = pl.pallas_call(
        functools.partial(_edge_kernel, hid, WB),
        grid=grid,
        in_specs=[
            pl.BlockSpec((TE, P), erow),
            pl.BlockSpec((TE, 1), erow),
            pl.BlockSpec(wre.shape, wfull),
            pl.BlockSpec(p['We2'].shape, wfull),
            pl.BlockSpec((1, hid), wfull),
            pl.BlockSpec(p['Wc1'].shape, wfull),
            pl.BlockSpec((1, hid), wfull),
            pl.BlockSpec(p['Wc2'].shape, wfull),
        ],
        out_specs=[
            pl.BlockSpec((TE, WB), lambda i: (i, 0)),
        ],
        out_shape=[
            jax.ShapeDtypeStruct((E, WB), jnp.float32),
        ],
        interpret=_INTERPRET,
    )(g, edge_attr,
      wre, p['We2'], p['be2'][None, :], p['Wc1'], p['bc1'][None, :], p['Wc2'])
    return out[0]


def _segment_sum(data, seg, num):
    return jax.ops.segment_sum(data, seg, num_segments=num)


def kernel(pos, edge_attr, params, edge_index, face, vertex2face, batch, ptr,
           face_len, vertex2face_len):
    row, col = edge_index[0], edge_index[1]

    centroid = jnp.mean(pos, axis=0, keepdims=True)
    p = pos - centroid
    mx = jnp.max(jnp.sqrt(jnp.sum(p ** 2, axis=1)))
    p = p / mx

    v0 = p[face[0]]
    v1 = p[face[1]]
    v2 = p[face[2]]
    fn = jnp.cross(v1 - v0, v2 - v0)
    face_area = jnp.sqrt(jnp.sum(fn ** 2, axis=1)) / 2.0
    vtx = vertex2face[:, 0]
    aval = jnp.concatenate([face_area, face_area, face_area])
    asum = _segment_sum(aval, vtx, N)
    acnt = jnp.maximum(_segment_sum(jnp.ones((vtx.shape[0],), jnp.float32), vtx, N), 1.0)
    area = asum / acnt
    x = area[:, None] * params['feat_W'][0][None, :] + params['feat_b'][None, :]

    coord = p
    for lp in (params['c1'], params['c2'], params['c3']):
        hid = lp['We2'].shape[0]
        inf = (lp['We1'].shape[0] - 2) // 2
        A = lp['We1'][:inf]
        B = lp['We1'][inf:2 * inf]
        P = 128
        pad = jnp.zeros((N, P - hid - 3), jnp.float32)
        tab_r = jnp.concatenate([x @ A + lp['be1'][None, :], coord, pad], axis=1)
        tab_c = jnp.concatenate([x @ B, -coord, pad], axis=1)
        g = _gather_add(tab_r, tab_c, row, col)
        WB = hid + 4
        mt = _edge_mlp(g, edge_attr, lp, WB)
        agg = _segment_sum(mt, row, N)
        magg = agg[:, :hid]
        trans = agg[:, hid:hid + 3]
        cnt = jnp.maximum(agg[:, hid + 3], 1.0)
        coord = coord + trans / cnt[:, None]
        h = jnp.concatenate([x, magg], axis=1)
        h = _silu(h @ lp['Wn1'] + lp['bn1'])
        x = h @ lp['Wn2'] + lp['bn2']

    x = jax.nn.relu(x @ params['lin1_W'] + params['lin1_b'])
    x = jnp.mean(x, axis=0, keepdims=True)
    x = x @ params['lin2_W'] + params['lin2_b']
    return jax.nn.log_softmax(x, axis=1)
